# baseline (speedup 1.0000x reference)
import functools

import jax
import jax.numpy as jnp
from jax import lax
from jax.experimental import pallas as pl
from jax.experimental.pallas import tpu as pltpu
from jax.experimental.pallas import tpu_sc as plsc

N = 10000
E = 320000
D = 128
EPS = 1e-5

NC = 2
NS = 16
NW = NC * NS
SCH = 128
ROWS = E // SCH
SROWS = 80
HROWS = SROWS // 2
LROWS = ROWS - 31 * SROWS
RPT = 624
TAIL = N - RPT * NS
TAIL_OFF = RPT * NS
MR = 128

def _zero_fill(buf, rows, width):
    def body(i, _):
        for q in range(width // 16):
            buf[i, pl.ds(q * 16, 16)] = jnp.zeros((16,), jnp.float32)
        return 0
    lax.fori_loop(0, rows, body, 0, unroll=False)


def _deg_body(dst3_hbm, ei2_hbm, degp_hbm, hist_v, didx_v):
    c = lax.axis_index("c")
    s = lax.axis_index("s")
    w = c * NS + s

    def zf(i, _):
        hist_v[pl.ds(16 * i, 16)] = jnp.zeros((16,), jnp.float32)
        return 0
    lax.fori_loop(0, N // 16, zf, 0, unroll=8)

    ones = jnp.ones((16,), jnp.float32)

    def step(r, _):
        for q in range(SCH // 16):
            idx = didx_v[r, pl.ds(16 * q, 16)]
            plsc.addupdate_scatter(hist_v, [idx], ones)
        return 0

    @pl.when(w < NW - 1)
    def _full():
        pltpu.sync_copy(dst3_hbm.at[pl.ds(w * SROWS, SROWS)], didx_v)
        lax.fori_loop(0, SROWS, step, 0, unroll=2)

    @pl.when(w == NW - 1)
    def _last():
        pltpu.sync_copy(
            dst3_hbm.at[pl.ds(w * SROWS, 16)], didx_v.at[pl.ds(0, 16)])
        for j in range(4):
            pltpu.sync_copy(
                ei2_hbm.at[1, pl.ds(E - 512 + 128 * j, SCH)],
                didx_v.at[16 + j])
        lax.fori_loop(0, LROWS, step, 0, unroll=2)

    pltpu.sync_copy(hist_v, degp_hbm.at[w])


@functools.lru_cache(maxsize=None)
def _deg_kernel():
    mesh = plsc.VectorSubcoreMesh(
        core_axis_name="c", subcore_axis_name="s", num_cores=NC, num_subcores=NS
    )
    return pl.kernel(
        _deg_body,
        out_type=jax.ShapeDtypeStruct((NW, N), jnp.float32),
        mesh=mesh,
        scratch_types=[
            pltpu.VMEM((N,), jnp.float32),
            pltpu.VMEM((SROWS, SCH), jnp.int32),
        ],
        compiler_params=pltpu.CompilerParams(needs_layout_passes=False),
    )


def _seg_zero(acc_sh, msga_v, s):
    _zero_fill(msga_v, MR, D)
    for k in range(5):
        pltpu.sync_copy(msga_v, acc_sh.at[pl.ds(s * RPT + k * MR, MR)])


def _seg_body(g_hbm, src3_hbm, dst3_hbm, ei2_hbm, out_hbm, acc_sh, msga_v,
              msgb_v, sidx_v, didx_v, sema, semb, semsa, semsb):
    c = lax.axis_index("c")
    s = lax.axis_index("s")
    w = c * NS + s

    _seg_zero(acc_sh, msga_v, s)
    plsc.subcore_barrier()

    def load2d(base_row, nrows, buf_row):
        pltpu.sync_copy(src3_hbm.at[pl.ds(base_row, nrows)],
                        sidx_v.at[pl.ds(buf_row, nrows)])
        pltpu.sync_copy(dst3_hbm.at[pl.ds(base_row, nrows)],
                        didx_v.at[pl.ds(buf_row, nrows)])

    def pipe(row0, nrows):
        pltpu.async_copy(g_hbm.at[sidx_v.at[row0]], msga_v, sema)
        pltpu.async_copy(g_hbm.at[sidx_v.at[row0 + 1]], msgb_v, semb)

        def step(j, _):
            a = row0 + 2 * j
            pltpu.make_async_copy(g_hbm.at[sidx_v.at[0]], msga_v, sema).wait()
            pltpu.async_copy(msga_v, acc_sh.at[didx_v.at[a]], semsa, add=True)
            pltpu.make_async_copy(g_hbm.at[sidx_v.at[0]], msgb_v, semb).wait()
            pltpu.async_copy(msgb_v, acc_sh.at[didx_v.at[a + 1]], semsb,
                             add=True)

            @pl.when(j < nrows // 2 - 1)
            def _prefetch():
                pltpu.make_async_copy(
                    msga_v, acc_sh.at[didx_v.at[0]], semsa).wait()
                pltpu.async_copy(g_hbm.at[sidx_v.at[a + 2]], msga_v, sema)
                pltpu.make_async_copy(
                    msgb_v, acc_sh.at[didx_v.at[0]], semsb).wait()
                pltpu.async_copy(g_hbm.at[sidx_v.at[a + 3]], msgb_v, semb)
            return 0
        lax.fori_loop(0, nrows // 2, step, 0, unroll=False)
        pltpu.make_async_copy(msga_v, acc_sh.at[didx_v.at[0]], semsa).wait()
        pltpu.make_async_copy(msgb_v, acc_sh.at[didx_v.at[0]], semsb).wait()

    @pl.when(w < NW - 1)
    def _full():
        for half in range(2):
            load2d(w * SROWS + half * HROWS, HROWS, 0)
            pipe(0, HROWS)

    @pl.when(w == NW - 1)
    def _last():
        load2d(w * SROWS, 16, 0)
        for j in range(4):
            pltpu.sync_copy(
                ei2_hbm.at[0, pl.ds(E - 512 + 128 * j, SCH)], sidx_v.at[16 + j])
            pltpu.sync_copy(
                ei2_hbm.at[1, pl.ds(E - 512 + 128 * j, SCH)], didx_v.at[16 + j])
        pipe(0, 16)
        pipe(16, 4)

    plsc.subcore_barrier()
    pltpu.sync_copy(
        acc_sh.at[pl.ds(s * RPT, RPT)], out_hbm.at[c, pl.ds(s * RPT, RPT)]
    )

    @pl.when(s == 0)
    def _otail():
        pltpu.sync_copy(
            acc_sh.at[pl.ds(TAIL_OFF, TAIL)], out_hbm.at[c, pl.ds(TAIL_OFF, TAIL)]
        )


@functools.lru_cache(maxsize=None)
def _seg_kernel():
    mesh = plsc.VectorSubcoreMesh(
        core_axis_name="c", subcore_axis_name="s", num_cores=NC, num_subcores=NS
    )
    return pl.kernel(
        _seg_body,
        out_type=jax.ShapeDtypeStruct((NC, N, D), jnp.float32),
        mesh=mesh,
        scratch_types=[
            pltpu.VMEM_SHARED((N, D), jnp.float32),
            pltpu.VMEM((MR, D), jnp.float32),
            pltpu.VMEM((MR, D), jnp.float32),
            pltpu.VMEM((HROWS, SCH), jnp.int32),
            pltpu.VMEM((HROWS, SCH), jnp.int32),
            pltpu.SemaphoreType.DMA,
            pltpu.SemaphoreType.DMA,
            pltpu.SemaphoreType.DMA,
            pltpu.SemaphoreType.DMA,
        ],
    )

BLK = 2000
NB = N // BLK


def _lin_body(x_ref, w_ref, b_ref, degp_ref, g_ref, dinv_ref):
    ones = jnp.ones((NW, 1), jnp.float32)
    tot = lax.dot_general(degp_ref[...], ones, (((0,), (0,)), ((), ())),
                          preferred_element_type=jnp.float32)
    dinv = lax.rsqrt(tot + 1.0)
    h = jnp.dot(x_ref[...], w_ref[...], preferred_element_type=jnp.float32)
    g_ref[...] = (h + b_ref[...]) * dinv
    dinv_ref[...] = dinv


_lin_call = pl.pallas_call(
    _lin_body,
    out_shape=(
        jax.ShapeDtypeStruct((N, D), jnp.float32),
        jax.ShapeDtypeStruct((N, 1), jnp.float32),
    ),
)


def _fin_body(sp_ref, g_ref, dinv_ref, gam_ref, bet_ref, out_ref, y_sc,
              sum_sc, sq_sc):
    p = pl.program_id(0)
    i = pl.program_id(1)

    @pl.when(p == 0)
    def _phase0():
        sp = sp_ref[...]
        y = jnp.tanh(dinv_ref[...] * (sp[0] + sp[1] + g_ref[...]))
        y_sc[pl.ds(i * BLK, BLK), :] = y

        @pl.when(i == 0)
        def _init():
            sum_sc[...] = jnp.zeros_like(sum_sc)
            sq_sc[...] = jnp.zeros_like(sq_sc)

        sum_sc[...] += jnp.sum(y, axis=0, keepdims=True)
        sq_sc[...] += jnp.sum(y * y, axis=0, keepdims=True)

    @pl.when(p == 1)
    def _phase1():
        mean = sum_sc[...] * (1.0 / N)
        var = sq_sc[...] * (1.0 / N) - mean * mean
        rstd = lax.rsqrt(var + EPS)
        y = y_sc[pl.ds(i * BLK, BLK), :]
        out_ref[...] = (y - mean) * (rstd * gam_ref[...]) + bet_ref[...]


_fin_call = pl.pallas_call(
    _fin_body,
    grid=(2, NB),
    in_specs=[
        pl.BlockSpec((2, BLK, D), lambda p, i: (0, i * (1 - p) + (NB - 1) * p, 0)),
        pl.BlockSpec((BLK, D), lambda p, i: (i * (1 - p) + (NB - 1) * p, 0)),
        pl.BlockSpec((BLK, 1), lambda p, i: (i * (1 - p) + (NB - 1) * p, 0)),
        pl.BlockSpec((1, D), lambda p, i: (0, 0)),
        pl.BlockSpec((1, D), lambda p, i: (0, 0)),
    ],
    out_specs=pl.BlockSpec((BLK, D), lambda p, i: (i, 0)),
    out_shape=jax.ShapeDtypeStruct((N, D), jnp.float32),
    scratch_shapes=[
        pltpu.VMEM((N, D), jnp.float32),
        pltpu.VMEM((1, D), jnp.float32),
        pltpu.VMEM((1, D), jnp.float32),
    ],
)


@jax.jit
def kernel(x, edge_index, W, b, gamma, beta):
    src3 = edge_index[0].reshape(ROWS, SCH)
    dst3 = edge_index[1].reshape(ROWS, SCH)
    degp = _deg_kernel()(dst3, edge_index)
    g, dinv = _lin_call(x, W, b.reshape(1, D), degp)
    sparts = _seg_kernel()(g, src3, dst3, edge_index)
    out = _fin_call(sparts, g, dinv, gamma.reshape(1, D), beta.reshape(1, D))
    return out


# speedup vs baseline: 1.2161x; 1.2161x over previous
import functools

import jax
import jax.numpy as jnp
from jax import lax
from jax.experimental import pallas as pl
from jax.experimental.pallas import tpu as pltpu
from jax.experimental.pallas import tpu_sc as plsc

N = 10000
E = 320000
D = 128
EPS = 1e-5

NC = 2
NS = 16
NW = NC * NS
SCH = 128
ROWS = E // SCH
SROWS = 80
HROWS = SROWS // 2
LROWS = ROWS - 31 * SROWS
RPT = 624
TAIL = N - RPT * NS
TAIL_OFF = RPT * NS
MR = 128

def _zero_fill(buf, rows, width):
    def body(i, _):
        for q in range(width // 16):
            buf[i, pl.ds(q * 16, 16)] = jnp.zeros((16,), jnp.float32)
        return 0
    lax.fori_loop(0, rows, body, 0, unroll=False)


def _deg_body(dst3_hbm, ei2_hbm, degp_hbm, hist_v, didx_v):
    c = lax.axis_index("c")
    s = lax.axis_index("s")
    w = c * NS + s

    def zf(i, _):
        hist_v[pl.ds(16 * i, 16)] = jnp.zeros((16,), jnp.float32)
        return 0
    lax.fori_loop(0, N // 16, zf, 0, unroll=8)

    ones = jnp.ones((16,), jnp.float32)

    def step(r, _):
        for q in range(SCH // 16):
            idx = didx_v[r, pl.ds(16 * q, 16)]
            plsc.addupdate_scatter(hist_v, [idx], ones)
        return 0

    @pl.when(w < NW - 1)
    def _full():
        pltpu.sync_copy(dst3_hbm.at[pl.ds(w * SROWS, SROWS)], didx_v)
        lax.fori_loop(0, SROWS, step, 0, unroll=2)

    @pl.when(w == NW - 1)
    def _last():
        pltpu.sync_copy(
            dst3_hbm.at[pl.ds(w * SROWS, 16)], didx_v.at[pl.ds(0, 16)])
        for j in range(4):
            pltpu.sync_copy(
                ei2_hbm.at[1, pl.ds(E - 512 + 128 * j, SCH)],
                didx_v.at[16 + j])
        lax.fori_loop(0, LROWS, step, 0, unroll=2)

    pltpu.sync_copy(hist_v, degp_hbm.at[w])


@functools.lru_cache(maxsize=None)
def _deg_kernel():
    mesh = plsc.VectorSubcoreMesh(
        core_axis_name="c", subcore_axis_name="s", num_cores=NC, num_subcores=NS
    )
    return pl.kernel(
        _deg_body,
        out_type=jax.ShapeDtypeStruct((NW, N), jnp.float32),
        mesh=mesh,
        scratch_types=[
            pltpu.VMEM((N,), jnp.float32),
            pltpu.VMEM((SROWS, SCH), jnp.int32),
        ],
        compiler_params=pltpu.CompilerParams(needs_layout_passes=False),
    )


def _seg_zero(acc_sh, msga_v, s):
    _zero_fill(msga_v, MR, D)
    for k in range(5):
        pltpu.sync_copy(msga_v, acc_sh.at[pl.ds(s * RPT + k * MR, MR)])


def _seg_body(g_hbm, src3_hbm, dst3_hbm, ei2_hbm, out_hbm, acc_sh, msga_v,
              msgb_v, sidx_v, didx_v, sema, semb, semsa, semsb):
    c = lax.axis_index("c")
    s = lax.axis_index("s")
    w = c * NS + s

    _seg_zero(acc_sh, msga_v, s)
    plsc.subcore_barrier()

    def load2d(base_row, nrows, buf_row):
        pltpu.sync_copy(src3_hbm.at[pl.ds(base_row, nrows)],
                        sidx_v.at[pl.ds(buf_row, nrows)])
        pltpu.sync_copy(dst3_hbm.at[pl.ds(base_row, nrows)],
                        didx_v.at[pl.ds(buf_row, nrows)])

    def pipe(row0, nrows):
        pltpu.async_copy(g_hbm.at[sidx_v.at[row0]], msga_v, sema)

        def step(j, _):
            a = row0 + 2 * j
            pltpu.async_copy(g_hbm.at[sidx_v.at[a + 1]], msgb_v, semb)
            pltpu.make_async_copy(g_hbm.at[sidx_v.at[0]], msga_v, sema).wait()
            pltpu.sync_copy(msga_v, acc_sh.at[didx_v.at[a]], add=True)

            @pl.when(j < nrows // 2 - 1)
            def _nexta():
                pltpu.async_copy(g_hbm.at[sidx_v.at[a + 2]], msga_v, sema)

            pltpu.make_async_copy(g_hbm.at[sidx_v.at[0]], msgb_v, semb).wait()
            pltpu.sync_copy(msgb_v, acc_sh.at[didx_v.at[a + 1]], add=True)
            return 0
        lax.fori_loop(0, nrows // 2, step, 0, unroll=False)

    @pl.when(w < NW - 1)
    def _full():
        for half in range(2):
            load2d(w * SROWS + half * HROWS, HROWS, 0)
            pipe(0, HROWS)

    @pl.when(w == NW - 1)
    def _last():
        load2d(w * SROWS, 16, 0)
        for j in range(4):
            pltpu.sync_copy(
                ei2_hbm.at[0, pl.ds(E - 512 + 128 * j, SCH)], sidx_v.at[16 + j])
            pltpu.sync_copy(
                ei2_hbm.at[1, pl.ds(E - 512 + 128 * j, SCH)], didx_v.at[16 + j])
        pipe(0, 16)
        pipe(16, 4)

    plsc.subcore_barrier()
    pltpu.sync_copy(
        acc_sh.at[pl.ds(s * RPT, RPT)], out_hbm.at[c, pl.ds(s * RPT, RPT)]
    )

    @pl.when(s == 0)
    def _otail():
        pltpu.sync_copy(
            acc_sh.at[pl.ds(TAIL_OFF, TAIL)], out_hbm.at[c, pl.ds(TAIL_OFF, TAIL)]
        )


@functools.lru_cache(maxsize=None)
def _seg_kernel():
    mesh = plsc.VectorSubcoreMesh(
        core_axis_name="c", subcore_axis_name="s", num_cores=NC, num_subcores=NS
    )
    return pl.kernel(
        _seg_body,
        out_type=jax.ShapeDtypeStruct((NC, N, D), jnp.float32),
        mesh=mesh,
        scratch_types=[
            pltpu.VMEM_SHARED((N, D), jnp.float32),
            pltpu.VMEM((MR, D), jnp.float32),
            pltpu.VMEM((MR, D), jnp.float32),
            pltpu.VMEM((HROWS, SCH), jnp.int32),
            pltpu.VMEM((HROWS, SCH), jnp.int32),
            pltpu.SemaphoreType.DMA,
            pltpu.SemaphoreType.DMA,
            pltpu.SemaphoreType.DMA,
            pltpu.SemaphoreType.DMA,
        ],
    )

BLK = 2000
NB = N // BLK


def _lin_body(x_ref, w_ref, b_ref, degp_ref, g_ref, dinv_ref):
    ones = jnp.ones((NW, 1), jnp.float32)
    tot = lax.dot_general(degp_ref[...], ones, (((0,), (0,)), ((), ())),
                          preferred_element_type=jnp.float32)
    dinv = lax.rsqrt(tot + 1.0)
    h = jnp.dot(x_ref[...], w_ref[...], preferred_element_type=jnp.float32)
    g_ref[...] = (h + b_ref[...]) * dinv
    dinv_ref[...] = dinv


_lin_call = pl.pallas_call(
    _lin_body,
    out_shape=(
        jax.ShapeDtypeStruct((N, D), jnp.float32),
        jax.ShapeDtypeStruct((N, 1), jnp.float32),
    ),
)


def _fin_body(sp_ref, g_ref, dinv_ref, gam_ref, bet_ref, out_ref, y_sc,
              sum_sc, sq_sc):
    p = pl.program_id(0)
    i = pl.program_id(1)

    @pl.when(p == 0)
    def _phase0():
        sp = sp_ref[...]
        y = jnp.tanh(dinv_ref[...] * (sp[0] + sp[1] + g_ref[...]))
        y_sc[pl.ds(i * BLK, BLK), :] = y

        @pl.when(i == 0)
        def _init():
            sum_sc[...] = jnp.zeros_like(sum_sc)
            sq_sc[...] = jnp.zeros_like(sq_sc)

        sum_sc[...] += jnp.sum(y, axis=0, keepdims=True)
        sq_sc[...] += jnp.sum(y * y, axis=0, keepdims=True)

    @pl.when(p == 1)
    def _phase1():
        mean = sum_sc[...] * (1.0 / N)
        var = sq_sc[...] * (1.0 / N) - mean * mean
        rstd = lax.rsqrt(var + EPS)
        y = y_sc[pl.ds(i * BLK, BLK), :]
        out_ref[...] = (y - mean) * (rstd * gam_ref[...]) + bet_ref[...]


_fin_call = pl.pallas_call(
    _fin_body,
    grid=(2, NB),
    in_specs=[
        pl.BlockSpec((2, BLK, D), lambda p, i: (0, i * (1 - p) + (NB - 1) * p, 0)),
        pl.BlockSpec((BLK, D), lambda p, i: (i * (1 - p) + (NB - 1) * p, 0)),
        pl.BlockSpec((BLK, 1), lambda p, i: (i * (1 - p) + (NB - 1) * p, 0)),
        pl.BlockSpec((1, D), lambda p, i: (0, 0)),
        pl.BlockSpec((1, D), lambda p, i: (0, 0)),
    ],
    out_specs=pl.BlockSpec((BLK, D), lambda p, i: (i, 0)),
    out_shape=jax.ShapeDtypeStruct((N, D), jnp.float32),
    scratch_shapes=[
        pltpu.VMEM((N, D), jnp.float32),
        pltpu.VMEM((1, D), jnp.float32),
        pltpu.VMEM((1, D), jnp.float32),
    ],
)


@jax.jit
def kernel(x, edge_index, W, b, gamma, beta):
    src3 = edge_index[0].reshape(ROWS, SCH)
    dst3 = edge_index[1].reshape(ROWS, SCH)
    degp = _deg_kernel()(dst3, edge_index)
    g, dinv = _lin_call(x, W, b.reshape(1, D), degp)
    sparts = _seg_kernel()(g, src3, dst3, edge_index)
    out = _fin_call(sparts, g, dinv, gamma.reshape(1, D), beta.reshape(1, D))
    return out


# speedup vs baseline: 1.3072x; 1.0749x over previous
import functools

import jax
import jax.numpy as jnp
from jax import lax
from jax.experimental import pallas as pl
from jax.experimental.pallas import tpu as pltpu
from jax.experimental.pallas import tpu_sc as plsc

N = 10000
E = 320000
D = 128
EPS = 1e-5

NC = 2
NS = 16
NW = NC * NS
SCH = 128
ROWS = E // SCH
SROWS = 80
HROWS = SROWS // 2
LROWS = ROWS - 31 * SROWS
RPT = 624
TAIL = N - RPT * NS
TAIL_OFF = RPT * NS
MR = 128

def _zero_fill(buf, rows, width):
    def body(i, _):
        for q in range(width // 16):
            buf[i, pl.ds(q * 16, 16)] = jnp.zeros((16,), jnp.float32)
        return 0
    lax.fori_loop(0, rows, body, 0, unroll=False)


def _deg_body(ei_hbm, ei2_hbm, degp_hbm, hist_v, didx_v):
    c = lax.axis_index("c")
    s = lax.axis_index("s")
    w = c * NS + s

    def zf(i, _):
        hist_v[pl.ds(16 * i, 16)] = jnp.zeros((16,), jnp.float32)
        return 0
    lax.fori_loop(0, N // 16, zf, 0, unroll=8)

    ones = jnp.ones((16,), jnp.float32)

    def step(r, _):
        for q in range(SCH // 16):
            idx = didx_v[r, pl.ds(16 * q, 16)]
            plsc.addupdate_scatter(hist_v, [idx], ones)
        return 0

    @pl.when(w < NW - 1)
    def _full():
        pltpu.sync_copy(ei_hbm.at[1, pl.ds(w * SROWS, SROWS)], didx_v)
        lax.fori_loop(0, SROWS, step, 0, unroll=2)

    @pl.when(w == NW - 1)
    def _last():
        pltpu.sync_copy(
            ei_hbm.at[1, pl.ds(w * SROWS, LROWS - 4)],
            didx_v.at[pl.ds(0, LROWS - 4)])
        for j in range(4):
            pltpu.sync_copy(
                ei2_hbm.at[1, pl.ds(E - 512 + 128 * j, SCH)],
                didx_v.at[LROWS - 4 + j])
        lax.fori_loop(0, LROWS, step, 0, unroll=2)

    pltpu.sync_copy(hist_v, degp_hbm.at[w])


@functools.lru_cache(maxsize=None)
def _deg_kernel():
    mesh = plsc.VectorSubcoreMesh(
        core_axis_name="c", subcore_axis_name="s", num_cores=NC, num_subcores=NS
    )
    return pl.kernel(
        _deg_body,
        out_type=jax.ShapeDtypeStruct((NW, N), jnp.float32),
        mesh=mesh,
        scratch_types=[
            pltpu.VMEM((N,), jnp.float32),
            pltpu.VMEM((SROWS, SCH), jnp.int32),
        ],
        compiler_params=pltpu.CompilerParams(needs_layout_passes=False),
    )


def _seg_zero(acc_sh, msga_v, s):
    _zero_fill(msga_v, MR, D)
    for k in range(5):
        pltpu.sync_copy(msga_v, acc_sh.at[pl.ds(s * RPT + k * MR, MR)])


def _seg_body(g_hbm, ei_hbm, ei2_hbm, out_hbm, acc_sh, msga_v, msgb_v,
              sidx_v, didx_v, sema, semb):
    c = lax.axis_index("c")
    s = lax.axis_index("s")
    w = c * NS + s

    _seg_zero(acc_sh, msga_v, s)
    plsc.subcore_barrier()

    def load2d(base_row, nrows):
        pltpu.sync_copy(
            ei_hbm.at[0, pl.ds(base_row, nrows)], sidx_v.at[pl.ds(0, nrows)])
        pltpu.sync_copy(
            ei_hbm.at[1, pl.ds(base_row, nrows)], didx_v.at[pl.ds(0, nrows)])

    def pipe(nrows):
        pltpu.async_copy(g_hbm.at[sidx_v.at[0]], msga_v, sema)

        def step(j, _):
            pltpu.async_copy(g_hbm.at[sidx_v.at[2 * j + 1]], msgb_v, semb)
            pltpu.make_async_copy(g_hbm.at[sidx_v.at[0]], msga_v, sema).wait()
            pltpu.sync_copy(msga_v, acc_sh.at[didx_v.at[2 * j]], add=True)

            @pl.when(j < nrows // 2 - 1)
            def _nexta():
                pltpu.async_copy(g_hbm.at[sidx_v.at[2 * j + 2]], msga_v, sema)

            pltpu.make_async_copy(g_hbm.at[sidx_v.at[0]], msgb_v, semb).wait()
            pltpu.sync_copy(msgb_v, acc_sh.at[didx_v.at[2 * j + 1]], add=True)
            return 0
        lax.fori_loop(0, nrows // 2, step, 0, unroll=False)

    @pl.when(w < NW - 1)
    def _full():
        for half in range(2):
            load2d(w * SROWS + half * HROWS, HROWS)
            pipe(HROWS)

    @pl.when(w == NW - 1)
    def _last():
        load2d(w * SROWS, LROWS - 4)
        for j in range(4):
            pltpu.sync_copy(
                ei2_hbm.at[0, pl.ds(E - 512 + 128 * j, SCH)],
                sidx_v.at[LROWS - 4 + j])
            pltpu.sync_copy(
                ei2_hbm.at[1, pl.ds(E - 512 + 128 * j, SCH)],
                didx_v.at[LROWS - 4 + j])
        pipe(LROWS)

    plsc.subcore_barrier()
    pltpu.sync_copy(
        acc_sh.at[pl.ds(s * RPT, RPT)], out_hbm.at[c, pl.ds(s * RPT, RPT)]
    )

    @pl.when(s == 0)
    def _otail():
        pltpu.sync_copy(
            acc_sh.at[pl.ds(TAIL_OFF, TAIL)], out_hbm.at[c, pl.ds(TAIL_OFF, TAIL)]
        )


@functools.lru_cache(maxsize=None)
def _seg_kernel():
    mesh = plsc.VectorSubcoreMesh(
        core_axis_name="c", subcore_axis_name="s", num_cores=NC, num_subcores=NS
    )
    return pl.kernel(
        _seg_body,
        out_type=jax.ShapeDtypeStruct((NC, N, D), jnp.float32),
        mesh=mesh,
        scratch_types=[
            pltpu.VMEM_SHARED((N, D), jnp.float32),
            pltpu.VMEM((MR, D), jnp.float32),
            pltpu.VMEM((MR, D), jnp.float32),
            pltpu.VMEM((HROWS, SCH), jnp.int32),
            pltpu.VMEM((HROWS, SCH), jnp.int32),
            pltpu.SemaphoreType.DMA,
            pltpu.SemaphoreType.DMA,
        ],
    )

BLK = 2000
NB = N // BLK


def _lin_body(x_ref, w_ref, b_ref, degp_ref, g_ref, dinv_ref):
    ones = jnp.ones((NW, 1), jnp.float32)
    tot = lax.dot_general(degp_ref[...], ones, (((0,), (0,)), ((), ())),
                          preferred_element_type=jnp.float32)
    dinv = lax.rsqrt(tot + 1.0)
    h = jnp.dot(x_ref[...], w_ref[...], preferred_element_type=jnp.float32)
    g_ref[...] = (h + b_ref[...]) * dinv
    dinv_ref[...] = dinv


_lin_call = pl.pallas_call(
    _lin_body,
    out_shape=(
        jax.ShapeDtypeStruct((N, D), jnp.float32),
        jax.ShapeDtypeStruct((N, 1), jnp.float32),
    ),
)


def _fin_body(sp_ref, g_ref, dinv_ref, gam_ref, bet_ref, out_ref, y_sc,
              sum_sc, sq_sc):
    p = pl.program_id(0)
    i = pl.program_id(1)

    @pl.when(p == 0)
    def _phase0():
        sp = sp_ref[...]
        y = jnp.tanh(dinv_ref[...] * (sp[0] + sp[1] + g_ref[...]))
        y_sc[pl.ds(i * BLK, BLK), :] = y

        @pl.when(i == 0)
        def _init():
            sum_sc[...] = jnp.zeros_like(sum_sc)
            sq_sc[...] = jnp.zeros_like(sq_sc)

        sum_sc[...] += jnp.sum(y, axis=0, keepdims=True)
        sq_sc[...] += jnp.sum(y * y, axis=0, keepdims=True)

    @pl.when(p == 1)
    def _phase1():
        mean = sum_sc[...] * (1.0 / N)
        var = sq_sc[...] * (1.0 / N) - mean * mean
        rstd = lax.rsqrt(var + EPS)
        y = y_sc[pl.ds(i * BLK, BLK), :]
        out_ref[...] = (y - mean) * (rstd * gam_ref[...]) + bet_ref[...]


_fin_call = pl.pallas_call(
    _fin_body,
    grid=(2, NB),
    in_specs=[
        pl.BlockSpec((2, BLK, D),
                     lambda p, i: (0, i * (1 - p) + (NB - 1) * p, 0)),
        pl.BlockSpec((BLK, D), lambda p, i: (i * (1 - p) + (NB - 1) * p, 0)),
        pl.BlockSpec((BLK, 1), lambda p, i: (i * (1 - p) + (NB - 1) * p, 0)),
        pl.BlockSpec((1, D), lambda p, i: (0, 0)),
        pl.BlockSpec((1, D), lambda p, i: (0, 0)),
    ],
    out_specs=pl.BlockSpec((BLK, D), lambda p, i: (i, 0)),
    out_shape=jax.ShapeDtypeStruct((N, D), jnp.float32),
    scratch_shapes=[
        pltpu.VMEM((N, D), jnp.float32),
        pltpu.VMEM((1, D), jnp.float32),
        pltpu.VMEM((1, D), jnp.float32),
    ],
)


@jax.jit
def kernel(x, edge_index, W, b, gamma, beta):
    ei = edge_index.reshape(2, ROWS, SCH)
    degp = _deg_kernel()(ei, edge_index)
    g, dinv = _lin_call(x, W, b.reshape(1, D), degp)
    sparts = _seg_kernel()(g, ei, edge_index)
    out = _fin_call(sparts, g, dinv, gamma.reshape(1, D), beta.reshape(1, D))
    return out
